# baseline (speedup 1.0000x reference)
import jax
import jax.numpy as jnp
from jax.experimental import pallas as pl
from jax.experimental.pallas import tpu as pltpu


_CHUNKS = (512, 256, 256)


def _pool_kernel(x_hbm, w_hbm, b_hbm, o_hbm, x_v, b_v, o_v, w_vmem, sems):
    offs = [0, _CHUNKS[0], _CHUNKS[0] + _CHUNKS[1]]

    def wcopy(i):
        return pltpu.make_async_copy(
            w_hbm.at[pl.ds(offs[i], _CHUNKS[i]), :],
            w_vmem.at[pl.ds(offs[i], _CHUNKS[i]), :],
            sems.at[i],
        )

    def xcopy():
        return pltpu.make_async_copy(
            x_hbm.at[:, pl.ds(0, 8), :], x_v, sems.at[3]
        )

    def bcopy():
        return pltpu.make_async_copy(b_hbm, b_v, sems.at[4])

    xcopy().start()
    bcopy().start()
    for i in range(len(_CHUNKS)):
        wcopy(i).start()

    xcopy().wait()
    bcopy().wait()
    x = x_v[:, 0, :]

    for i in range(len(_CHUNKS)):
        wcopy(i).wait()
        w = w_vmem[pl.ds(offs[i], _CHUNKS[i]), :]
        acc = jax.lax.dot_general(
            x, w,
            dimension_numbers=(((1,), (1,)), ((), ())),
            preferred_element_type=jnp.float32,
        )
        o_v[:, pl.ds(offs[i], _CHUNKS[i])] = jnp.tanh(
            acc + b_v[:, pl.ds(offs[i], _CHUNKS[i])]
        )

    ocopy = pltpu.make_async_copy(o_v, o_hbm, sems.at[5])
    ocopy.start()
    ocopy.wait()


@jax.jit
def kernel(hidden_states, W, b):
    B, _, H = hidden_states.shape
    O = W.shape[0]

    b2 = b.reshape(1, O)

    out = pl.pallas_call(
        _pool_kernel,
        in_specs=[
            pl.BlockSpec(memory_space=pl.ANY),
            pl.BlockSpec(memory_space=pl.ANY),
            pl.BlockSpec(memory_space=pl.ANY),
        ],
        out_specs=pl.BlockSpec(memory_space=pl.ANY),
        out_shape=jax.ShapeDtypeStruct((B, O), jnp.float32),
        scratch_shapes=[
            pltpu.VMEM((B, 8, H), jnp.float32),
            pltpu.VMEM((1, O), jnp.float32),
            pltpu.VMEM((B, O), jnp.float32),
            pltpu.VMEM((O, H), jnp.float32),
            pltpu.SemaphoreType.DMA((6,)),
        ],
    )(hidden_states, W, b2)
    return out


# speedup vs baseline: 1.0763x; 1.0763x over previous
import jax
import jax.numpy as jnp
from jax.experimental import pallas as pl
from jax.experimental.pallas import tpu as pltpu


_CHUNKS = (512, 512)


def _pool_kernel(x_hbm, w_hbm, b_hbm, o_hbm, x_v, b_v, o_v, w_vmem, sems):
    offs = [0, _CHUNKS[0]]

    def wcopy(i):
        return pltpu.make_async_copy(
            w_hbm.at[pl.ds(offs[i], _CHUNKS[i]), :],
            w_vmem.at[pl.ds(offs[i], _CHUNKS[i]), :],
            sems.at[i],
        )

    def xcopy():
        return pltpu.make_async_copy(
            x_hbm.at[:, pl.ds(0, 8), :], x_v, sems.at[2]
        )

    def bcopy():
        return pltpu.make_async_copy(b_hbm, b_v, sems.at[3])

    def ocopy(i):
        return pltpu.make_async_copy(
            o_v.at[:, pl.ds(offs[i], _CHUNKS[i])],
            o_hbm.at[:, pl.ds(offs[i], _CHUNKS[i])],
            sems.at[4 + i],
        )

    xcopy().start()
    bcopy().start()
    wcopy(0).start()
    wcopy(1).start()

    xcopy().wait()
    bcopy().wait()
    x = x_v[:, 0, :]

    for i in range(2):
        wcopy(i).wait()
        w = w_vmem[pl.ds(offs[i], _CHUNKS[i]), :]
        acc = jax.lax.dot_general(
            x, w,
            dimension_numbers=(((1,), (1,)), ((), ())),
            preferred_element_type=jnp.float32,
        )
        o_v[:, pl.ds(offs[i], _CHUNKS[i])] = jnp.tanh(
            acc + b_v[:, pl.ds(offs[i], _CHUNKS[i])]
        )
        ocopy(i).start()

    ocopy(0).wait()
    ocopy(1).wait()


@jax.jit
def kernel(hidden_states, W, b):
    B, _, H = hidden_states.shape
    O = W.shape[0]

    b2 = b.reshape(1, O)

    out = pl.pallas_call(
        _pool_kernel,
        in_specs=[
            pl.BlockSpec(memory_space=pl.ANY),
            pl.BlockSpec(memory_space=pl.ANY),
            pl.BlockSpec(memory_space=pl.ANY),
        ],
        out_specs=pl.BlockSpec(memory_space=pl.ANY),
        out_shape=jax.ShapeDtypeStruct((B, O), jnp.float32),
        scratch_shapes=[
            pltpu.VMEM((B, 8, H), jnp.float32),
            pltpu.VMEM((1, O), jnp.float32),
            pltpu.VMEM((B, O), jnp.float32),
            pltpu.VMEM((O, H), jnp.float32),
            pltpu.SemaphoreType.DMA((6,)),
        ],
    )(hidden_states, W, b2)
    return out
